# baseline (speedup 1.0000x reference)
import functools

import jax
import jax.numpy as jnp
from jax import lax
from jax.experimental import pallas as pl
from jax.experimental.pallas import tpu as pltpu
from jax.experimental.pallas import tpu_sc as plsc

NUM_EMB = 1_000_000
DIM = 64
BATCH = 4096
HIST = 200
NC, NS = 2, 16
NW = NC * NS
B_PER_W = BATCH // NW
HIST_PAD = 208
CHUNK = 40
K = HIST // CHUNK
T_TILES = HIST // 8


def _emb_body(wt_hbm, weight_hbm, out_hbm, stage_v, idx_v, rows_v, gsem, wsem):
    wid = lax.axis_index("s") * NC + lax.axis_index("c")
    b0 = wid * B_PER_W

    for ti in range(T_TILES):
        pltpu.sync_copy(
            wt_hbm.at[pl.ds(ti * (NW * 1024) + wid * 1024, 1024)],
            stage_v.at[pl.ds(ti * 1024, 1024)],
        )

    lanes = lax.iota(jnp.int32, 16) * 128

    def transpose_body(bb, carry):
        for t0 in range(0, HIST_PAD, 16):
            src = lanes + (t0 * 128 + bb)
            idx_v[pl.ds(bb * HIST_PAD + t0, 16)] = plsc.load_gather(
                stage_v, [src]
            )
        return carry

    lax.fori_loop(0, B_PER_W, transpose_body, 0)

    def gather_copy(g, buf, j):
        return pltpu.make_async_copy(
            weight_hbm.at[idx_v.at[pl.ds(g * HIST_PAD + j * CHUNK, CHUNK)]],
            rows_v.at[buf, pl.ds(j * CHUNK, CHUNK)],
            gsem.at[buf],
        )

    def start_group(g, buf):
        for j in range(K):
            gather_copy(g, buf, j).start()

    def wait_group(g, buf):
        for j in range(K):
            gather_copy(g, buf, j).wait()

    def write_copy(g, buf):
        return pltpu.make_async_copy(
            rows_v.at[buf],
            out_hbm.at[b0 + g],
            wsem.at[buf],
        )

    start_group(0, 0)
    start_group(1, 1)

    def body(i, carry):
        g = 2 * i
        for buf in (0, 1):
            wait_group(g + buf, buf)
            write_copy(g + buf, buf).start()
            write_copy(g + buf, buf).wait()
            start_group(g + buf + 2, buf)
        return carry

    lax.fori_loop(0, (B_PER_W - 2) // 2, body, 0)

    for buf in (0, 1):
        g = B_PER_W - 2 + buf
        wait_group(g, buf)
        write_copy(g, buf).start()
    for buf in (0, 1):
        write_copy(B_PER_W - 2 + buf, buf).wait()


@functools.partial(jax.jit)
def _embedding_gather(words_flat, weight):
    mesh = plsc.VectorSubcoreMesh(core_axis_name="c", subcore_axis_name="s")
    f = pl.kernel(
        _emb_body,
        out_type=jax.ShapeDtypeStruct((BATCH, HIST, DIM), jnp.float32),
        mesh=mesh,
        scratch_types=[
            pltpu.VMEM((B_PER_W * HIST_PAD,), jnp.int32),
            pltpu.VMEM((B_PER_W * HIST_PAD,), jnp.int32),
            pltpu.VMEM((2, HIST, DIM), jnp.float32),
            pltpu.SemaphoreType.DMA((2,)),
            pltpu.SemaphoreType.DMA((2,)),
        ],
        compiler_params=pltpu.CompilerParams(use_tc_tiling_on_sc=False, needs_layout_passes=False),
    )
    return f(words_flat, weight)


def kernel(words, weight):
    wt = words.T.reshape(T_TILES, 8, NW, B_PER_W)
    wt = wt.transpose(0, 2, 1, 3).reshape(-1).astype(jnp.int32)
    return _embedding_gather(wt, weight)
